# baseline (speedup 1.0000x reference)
import functools

import jax
import jax.numpy as jnp
from jax import lax
from jax.experimental import pallas as pl
from jax.experimental.pallas import tpu as pltpu
from jax.experimental.pallas import tpu_sc as plsc

N = 50000
E = 800000
EPS = 1e-7

NW = 32
B = 256
W = 64
EPAD = E + 512
NEG = -1e30


def _enc3_kernel(x_ref, w0, b0, w1, b1, w2, b2, o_ref):
    h = jnp.tanh(x_ref[...] @ w0[...] + b0[...])
    h = jnp.tanh(h @ w1[...] + b1[...])
    o_ref[...] = jnp.tanh(h @ w2[...] + b2[...])


def _encoder3(x, layers, rows):
    n, din = x.shape
    grid = n // rows
    return pl.pallas_call(
        _enc3_kernel,
        out_shape=jax.ShapeDtypeStruct((n, 64), jnp.float32),
        grid=(grid,),
        in_specs=[
            pl.BlockSpec((rows, din), lambda i: (i, 0)),
            pl.BlockSpec(layers[0]["W"].shape, lambda i: (0, 0)),
            pl.BlockSpec((64,), lambda i: (0,)),
            pl.BlockSpec((64, 64), lambda i: (0, 0)),
            pl.BlockSpec((64,), lambda i: (0,)),
            pl.BlockSpec((64, 64), lambda i: (0, 0)),
            pl.BlockSpec((64,), lambda i: (0,)),
        ],
        out_specs=pl.BlockSpec((rows, 64), lambda i: (i, 0)),
    )(x, layers[0]["W"], layers[0]["b"], layers[1]["W"], layers[1]["b"],
      layers[2]["W"], layers[2]["b"])


def _block_mlp_kernel(aggr_ref, x_ref, w0, lnw, lnb, w4, o_ref):
    out = aggr_ref[...] + x_ref[...]
    h = out @ w0[...]
    mu = jnp.mean(h, axis=-1, keepdims=True)
    var = jnp.mean((h - mu) * (h - mu), axis=-1, keepdims=True)
    h = (h - mu) / jnp.sqrt(var + 1e-5) * lnw[...] + lnb[...]
    h = jnp.maximum(h, 0.0)
    o_ref[...] = h @ w4[...]


def _block_mlp(aggr, x, p, rows=2000):
    grid = N // rows
    return pl.pallas_call(
        _block_mlp_kernel,
        out_shape=jax.ShapeDtypeStruct((N, 64), jnp.float32),
        grid=(grid,),
        in_specs=[
            pl.BlockSpec((rows, 64), lambda i: (i, 0)),
            pl.BlockSpec((rows, 64), lambda i: (i, 0)),
            pl.BlockSpec((64, 128), lambda i: (0, 0)),
            pl.BlockSpec((128,), lambda i: (0,)),
            pl.BlockSpec((128,), lambda i: (0,)),
            pl.BlockSpec((128, 64), lambda i: (0, 0)),
        ],
        out_specs=pl.BlockSpec((rows, 64), lambda i: (i, 0)),
    )(aggr, x, p["W0"], p["ln_w"], p["ln_b"], p["W4"])


def _block3_head_kernel(aggr_ref, x_ref, w0, lnw, lnb, w4,
                        hw0, hb0, hw1, hb1, hw2, hb2, o_ref):
    out = aggr_ref[...] + x_ref[...]
    h = out @ w0[...]
    mu = jnp.mean(h, axis=-1, keepdims=True)
    var = jnp.mean((h - mu) * (h - mu), axis=-1, keepdims=True)
    h = (h - mu) / jnp.sqrt(var + 1e-5) * lnw[...] + lnb[...]
    h = jnp.maximum(h, 0.0)
    x3 = h @ w4[...]
    g = jnp.tanh(x3 @ hw0[...] + hb0[...])
    g = jnp.tanh(g @ hw1[...] + hb1[...])
    out0 = (g @ hw2[...])[:, 0] + hb2[...][0]
    o_ref[...] = (-30.0 + (out0 + 1.0) * 30.0)[None, None, :]


def _block3_head(aggr, x, p, pi, rows=2000):
    grid = N // rows
    out = pl.pallas_call(
        _block3_head_kernel,
        out_shape=jax.ShapeDtypeStruct((grid, 1, rows), jnp.float32),
        grid=(grid,),
        in_specs=[
            pl.BlockSpec((rows, 64), lambda i: (i, 0)),
            pl.BlockSpec((rows, 64), lambda i: (i, 0)),
            pl.BlockSpec((64, 128), lambda i: (0, 0)),
            pl.BlockSpec((128,), lambda i: (0,)),
            pl.BlockSpec((128,), lambda i: (0,)),
            pl.BlockSpec((128, 64), lambda i: (0, 0)),
            pl.BlockSpec((64, 64), lambda i: (0, 0)),
            pl.BlockSpec((64,), lambda i: (0,)),
            pl.BlockSpec((64, 64), lambda i: (0, 0)),
            pl.BlockSpec((64,), lambda i: (0,)),
            pl.BlockSpec((64, 2), lambda i: (0, 0)),
            pl.BlockSpec((2,), lambda i: (0,)),
        ],
        out_specs=pl.BlockSpec((1, 1, rows), lambda i: (i, 0, 0)),
    )(aggr, x, p["W0"], p["ln_w"], p["ln_b"], p["W4"],
      pi[0]["W"], pi[0]["b"], pi[1]["W"], pi[1]["b"], pi[2]["W"], pi[2]["b"])
    return out.reshape(N)


def _sc_aggregate_body(x_hbm, ee_hbm, dst_hbm, src_hbm, order_hbm,
                       starts_hbm, bases_hbm, out_hbm,
                       starts_v, bases_v, dst_v, src_v, order_v,
                       eeb, xg, p_buf, mp_buf, n_win, d_win, acc_v,
                       semi, semg):
    c = lax.axis_index("c")
    s = lax.axis_index("s")
    w = s * 2 + c

    pltpu.sync_copy(starts_hbm, starts_v)
    pltpu.sync_copy(bases_hbm, bases_v)
    start = starts_v[pl.ds(w, 16)][0]
    end = starts_v[pl.ds(w + 1, 16)][0]
    lo_node = bases_v[pl.ds(w, 16)][0]
    hi_node = bases_v[pl.ds(w + 1, 16)][0]

    base0 = start & ~7
    nblk = (end - base0 + B - 1) // B

    zeros16 = jnp.zeros((16,), jnp.float32)

    def zero_win(r, _):
        for j in range(4):
            n_win[r, pl.ds(16 * j, 16)] = zeros16
            d_win[r, pl.ds(16 * j, 16)] = zeros16
        return 0

    lax.fori_loop(0, W, zero_win, 0)
    for j in range(8):
        acc_v[j, :] = zeros16

    def advance(cur_base, target):
        nfl = jnp.maximum((target - cur_base) // W, 0)

        def div_row(r, _):
            for j in range(4):
                nv = n_win[r, pl.ds(16 * j, 16)]
                dv = d_win[r, pl.ds(16 * j, 16)]
                n_win[r, pl.ds(16 * j, 16)] = nv / (dv + 1e-33)
            return 0

        def flush_i(i, cb):
            lax.fori_loop(0, W, div_row, 0)
            pltpu.sync_copy(n_win, out_hbm.at[pl.ds(cb, W)])
            lax.fori_loop(0, W, zero_win, 0)
            return cb + W

        return lax.fori_loop(0, nfl, flush_i, cur_base)

    def issue_idx(k):
        slot = k % 2
        base = pl.multiple_of(base0 + k * B, 8)
        pltpu.async_copy(dst_hbm.at[pl.ds(base, B)],
                         dst_v.at[slot, pl.ds(0, B)], semi)
        pltpu.async_copy(src_hbm.at[pl.ds(base, B)], src_v.at[slot], semi)
        pltpu.async_copy(order_hbm.at[pl.ds(base, B)], order_v.at[slot], semi)

    def wait_idx():
        for _ in range(2):
            pltpu.make_async_copy(src_hbm.at[pl.ds(0, B)],
                                  src_v.at[0], semi).wait()
        pltpu.make_async_copy(dst_hbm.at[pl.ds(0, B)],
                              dst_v.at[0, pl.ds(0, B)], semi).wait()

    def issue_gather(k):
        slot = k % 2
        pltpu.async_copy(ee_hbm.at[order_v.at[slot]], eeb.at[slot], semg)
        pltpu.async_copy(x_hbm.at[src_v.at[slot]], xg.at[slot], semg)

    def wait_gather():
        for _ in range(2):
            pltpu.make_async_copy(ee_hbm.at[pl.ds(0, B)],
                                  eeb.at[0], semg).wait()

    def blk_fn(k, carry):
        slot = k % 2
        base = base0 + k * B
        lo = jnp.maximum(start - base, 0)
        hi = jnp.minimum(end - base, B)
        wait_gather()

        def prep_fn(r, _):
            for j in range(4):
                xs = xg[slot, r, pl.ds(16 * j, 16)]
                es = eeb[slot, r, pl.ds(16 * j, 16)]
                msg = jnp.maximum(xs + es, 0.0) + EPS
                p = jnp.exp(msg - 60.0)
                p_buf[r, pl.ds(16 * j, 16)] = p
                mp_buf[r, pl.ds(16 * j, 16)] = msg * p
            return 0

        lax.fori_loop(0, B, prep_fn, 0)

        @pl.when(k + 1 < nblk)
        def _():
            wait_idx()
            issue_gather(k + 1)

        def edge_fn(e, carry2):
            (cur_base, prev, d0, d1, d2, d3, n0, n1, n2, n3) = carry2
            d_e = dst_v[slot, pl.ds(e, 16)][0]
            is_new = d_e != prev
            cur_base = advance(cur_base, d_e)
            keep = jnp.where(is_new, 0.0, 1.0)
            ds = [d0, d1, d2, d3]
            ns = [n0, n1, n2, n3]
            row = d_e - cur_base
            for j in range(4):
                p = p_buf[e, pl.ds(16 * j, 16)]
                mp = mp_buf[e, pl.ds(16 * j, 16)]
                d_j = ds[j] * keep + p
                n_j = ns[j] * keep + mp
                n_win[row, pl.ds(16 * j, 16)] = n_j
                d_win[row, pl.ds(16 * j, 16)] = d_j
                ds[j] = d_j
                ns[j] = n_j
            return (cur_base, d_e, ds[0], ds[1], ds[2], ds[3],
                    ns[0], ns[1], ns[2], ns[3])

        def chunk_fn(ch, carry2):
            e0 = ch * 16
            dvec = dst_v[slot, pl.ds(e0, 16)]
            cur_base = carry2[0]
            in_range = jnp.logical_and(lo <= e0, e0 + 16 <= hi)
            fast = jnp.logical_and(in_range, dvec[15] < cur_base + W)

            def fast_fn(args):
                cur_base, prev = args
                ds = [acc_v[j, :] for j in range(4)]
                ns = [acc_v[4 + j, :] for j in range(4)]
                pr = prev
                for i in range(16):
                    d_e = dvec[i]
                    keep = jnp.where(d_e == pr, 1.0, 0.0)
                    row = d_e - cur_base
                    for j in range(4):
                        p = p_buf[e0 + i, pl.ds(16 * j, 16)]
                        mp = mp_buf[e0 + i, pl.ds(16 * j, 16)]
                        d_j = ds[j] * keep + p
                        n_j = ns[j] * keep + mp
                        n_win[row, pl.ds(16 * j, 16)] = n_j
                        d_win[row, pl.ds(16 * j, 16)] = d_j
                        ds[j] = d_j
                        ns[j] = n_j
                    pr = d_e
                for j in range(4):
                    acc_v[j, :] = ds[j]
                    acc_v[4 + j, :] = ns[j]
                return (cur_base, pr)

            def slow_fn(args):
                cur_base, prev = args
                lo_e = jnp.maximum(lo, e0)
                hi_e = jnp.minimum(hi, e0 + 16)
                full = (cur_base, prev) + tuple(
                    acc_v[j, :] for j in range(8))
                full = lax.fori_loop(lo_e, hi_e, edge_fn, full)
                for j in range(8):
                    acc_v[j, :] = full[2 + j]
                return (full[0], full[1])

            return lax.cond(fast, fast_fn, slow_fn, carry2)

        carry = lax.fori_loop(0, B // 16, chunk_fn, carry)

        @pl.when(k + 2 < nblk)
        def _():
            issue_idx(k + 2)

        return carry

    init = (lo_node, jnp.int32(-1))

    @pl.when(nblk > 0)
    def _():
        issue_idx(0)
        wait_idx()
        issue_gather(0)

    @pl.when(nblk > 1)
    def _():
        issue_idx(1)

    carry = lax.fori_loop(0, nblk, blk_fn, init)
    cur_base = carry[0]

    cur_base = advance(cur_base, hi_node)
    nrows = hi_node - cur_base

    def tail_div(r, _):
        for j in range(4):
            nv = n_win[r, pl.ds(16 * j, 16)]
            dv = d_win[r, pl.ds(16 * j, 16)]
            n_win[r, pl.ds(16 * j, 16)] = nv / (dv + 1e-33)
        return 0

    lax.fori_loop(0, nrows, tail_div, 0)

    def tail_fire(r, _):
        pltpu.async_copy(n_win.at[pl.ds(r, 1)],
                         out_hbm.at[pl.ds(cur_base + r, 1)], semg)
        return 0

    def tail_drain(r, _):
        pltpu.make_async_copy(n_win.at[pl.ds(0, 1)],
                              out_hbm.at[pl.ds(cur_base, 1)], semg).wait()
        return 0

    lax.fori_loop(0, nrows, tail_fire, 0)
    lax.fori_loop(0, nrows, tail_drain, 0)


def _sc_aggregate(x, ee, dst_p, src_p, order_p, starts, bases):
    mesh = plsc.VectorSubcoreMesh(core_axis_name="c", subcore_axis_name="s")
    f = pl.kernel(
        _sc_aggregate_body,
        out_type=jax.ShapeDtypeStruct((N, 64), jnp.float32),
        mesh=mesh,
        compiler_params=pltpu.CompilerParams(use_tc_tiling_on_sc=False),
        scratch_types=[
            pltpu.VMEM((48,), jnp.int32),
            pltpu.VMEM((48,), jnp.int32),
            pltpu.VMEM((2, B + 16), jnp.int32),
            pltpu.VMEM((2, B), jnp.int32),
            pltpu.VMEM((2, B), jnp.int32),
            pltpu.VMEM((2, B, 64), jnp.float32),
            pltpu.VMEM((2, B, 64), jnp.float32),
            pltpu.VMEM((B, 64), jnp.float32),
            pltpu.VMEM((B, 64), jnp.float32),
            pltpu.VMEM((W, 64), jnp.float32),
            pltpu.VMEM((W, 64), jnp.float32),
            pltpu.VMEM((8, 16), jnp.float32),
            pltpu.SemaphoreType.DMA,
            pltpu.SemaphoreType.DMA,
        ],
    )
    return f(x, ee, dst_p, src_p, order_p, starts, bases)


def kernel(node_features, edge_features, edge_links, params):
    src = edge_links[0]
    dst = edge_links[1]

    order = jnp.argsort(dst).astype(jnp.int32)
    dst_s = jnp.take(dst, order)
    src_s = jnp.take(src, order)

    pad = EPAD - E
    dst_p = jnp.concatenate([dst_s, jnp.full((pad,), N, jnp.int32)])
    src_p = jnp.concatenate([src_s, jnp.zeros((pad,), jnp.int32)])
    order_p = jnp.concatenate([order, jnp.zeros((pad,), jnp.int32)])

    nominal = (jnp.arange(1, NW) * E) // NW
    dvals = jnp.take(dst_s, nominal)
    starts_mid = jnp.searchsorted(dst_s, dvals, side="left").astype(jnp.int32)
    starts = jnp.concatenate(
        [jnp.zeros((1,), jnp.int32), starts_mid,
         jnp.full((1,), E, jnp.int32), jnp.zeros((48 - NW - 1,), jnp.int32)])
    base_mid = jnp.take(dst_s, starts_mid)
    bases = jnp.concatenate(
        [jnp.zeros((1,), jnp.int32), base_mid,
         jnp.full((1,), N, jnp.int32), jnp.zeros((48 - NW - 1,), jnp.int32)])

    x = _encoder3(node_features, params["node_enc"], rows=2000)
    ee = _encoder3(edge_features, params["edge_enc"], rows=3200)

    for bi, p in enumerate(params["mp"]):
        aggr = _sc_aggregate(x, ee, dst_p, src_p, order_p, starts, bases)
        if bi < 2:
            x = _block_mlp(aggr, x, p)
        else:
            return _block3_head(aggr, x, p, params["pi"])


# speedup vs baseline: 2.0092x; 2.0092x over previous
import functools

import jax
import jax.numpy as jnp
from jax import lax
from jax.experimental import pallas as pl
from jax.experimental.pallas import tpu as pltpu
from jax.experimental.pallas import tpu_sc as plsc

N = 50000
E = 800000
EPS = 1e-7

NW = 32
B = 256
W = 64
EPAD = E + 512
NEG = -1e30


def _enc3_kernel(x_ref, w0, b0, w1, b1, w2, b2, o_ref):
    h = jnp.tanh(x_ref[...] @ w0[...] + b0[...])
    h = jnp.tanh(h @ w1[...] + b1[...])
    o_ref[...] = jnp.tanh(h @ w2[...] + b2[...])


def _encoder3(x, layers, rows):
    n, din = x.shape
    grid = n // rows
    return pl.pallas_call(
        _enc3_kernel,
        out_shape=jax.ShapeDtypeStruct((n, 64), jnp.float32),
        grid=(grid,),
        in_specs=[
            pl.BlockSpec((rows, din), lambda i: (i, 0)),
            pl.BlockSpec(layers[0]["W"].shape, lambda i: (0, 0)),
            pl.BlockSpec((64,), lambda i: (0,)),
            pl.BlockSpec((64, 64), lambda i: (0, 0)),
            pl.BlockSpec((64,), lambda i: (0,)),
            pl.BlockSpec((64, 64), lambda i: (0, 0)),
            pl.BlockSpec((64,), lambda i: (0,)),
        ],
        out_specs=pl.BlockSpec((rows, 64), lambda i: (i, 0)),
    )(x, layers[0]["W"], layers[0]["b"], layers[1]["W"], layers[1]["b"],
      layers[2]["W"], layers[2]["b"])


def _block_mlp_kernel(aggr_ref, x_ref, w0, lnw, lnb, w4, o_ref):
    out = aggr_ref[...] + x_ref[...]
    h = out @ w0[...]
    mu = jnp.mean(h, axis=-1, keepdims=True)
    var = jnp.mean((h - mu) * (h - mu), axis=-1, keepdims=True)
    h = (h - mu) / jnp.sqrt(var + 1e-5) * lnw[...] + lnb[...]
    h = jnp.maximum(h, 0.0)
    o_ref[...] = h @ w4[...]


def _block_mlp(aggr, x, p, rows=2000):
    grid = N // rows
    return pl.pallas_call(
        _block_mlp_kernel,
        out_shape=jax.ShapeDtypeStruct((N, 64), jnp.float32),
        grid=(grid,),
        in_specs=[
            pl.BlockSpec((rows, 64), lambda i: (i, 0)),
            pl.BlockSpec((rows, 64), lambda i: (i, 0)),
            pl.BlockSpec((64, 128), lambda i: (0, 0)),
            pl.BlockSpec((128,), lambda i: (0,)),
            pl.BlockSpec((128,), lambda i: (0,)),
            pl.BlockSpec((128, 64), lambda i: (0, 0)),
        ],
        out_specs=pl.BlockSpec((rows, 64), lambda i: (i, 0)),
    )(aggr, x, p["W0"], p["ln_w"], p["ln_b"], p["W4"])


def _block3_head_kernel(aggr_ref, x_ref, w0, lnw, lnb, w4,
                        hw0, hb0, hw1, hb1, hw2, hb2, o_ref):
    out = aggr_ref[...] + x_ref[...]
    h = out @ w0[...]
    mu = jnp.mean(h, axis=-1, keepdims=True)
    var = jnp.mean((h - mu) * (h - mu), axis=-1, keepdims=True)
    h = (h - mu) / jnp.sqrt(var + 1e-5) * lnw[...] + lnb[...]
    h = jnp.maximum(h, 0.0)
    x3 = h @ w4[...]
    g = jnp.tanh(x3 @ hw0[...] + hb0[...])
    g = jnp.tanh(g @ hw1[...] + hb1[...])
    out0 = (g @ hw2[...])[:, 0] + hb2[...][0]
    o_ref[...] = (-30.0 + (out0 + 1.0) * 30.0)[None, None, :]


def _block3_head(aggr, x, p, pi, rows=2000):
    grid = N // rows
    out = pl.pallas_call(
        _block3_head_kernel,
        out_shape=jax.ShapeDtypeStruct((grid, 1, rows), jnp.float32),
        grid=(grid,),
        in_specs=[
            pl.BlockSpec((rows, 64), lambda i: (i, 0)),
            pl.BlockSpec((rows, 64), lambda i: (i, 0)),
            pl.BlockSpec((64, 128), lambda i: (0, 0)),
            pl.BlockSpec((128,), lambda i: (0,)),
            pl.BlockSpec((128,), lambda i: (0,)),
            pl.BlockSpec((128, 64), lambda i: (0, 0)),
            pl.BlockSpec((64, 64), lambda i: (0, 0)),
            pl.BlockSpec((64,), lambda i: (0,)),
            pl.BlockSpec((64, 64), lambda i: (0, 0)),
            pl.BlockSpec((64,), lambda i: (0,)),
            pl.BlockSpec((64, 2), lambda i: (0, 0)),
            pl.BlockSpec((2,), lambda i: (0,)),
        ],
        out_specs=pl.BlockSpec((1, 1, rows), lambda i: (i, 0, 0)),
    )(aggr, x, p["W0"], p["ln_w"], p["ln_b"], p["W4"],
      pi[0]["W"], pi[0]["b"], pi[1]["W"], pi[1]["b"], pi[2]["W"], pi[2]["b"])
    return out.reshape(N)


def _sc_aggregate_body(x_hbm, ee_hbm, dst_hbm, src_hbm, order_hbm,
                       starts_hbm, bases_hbm, out_hbm,
                       starts_v, bases_v, dst_v0, src_v0, order_v0,
                       dst_v1, src_v1, order_v1, eeb0, xg0, eeb1, xg1,
                       p_buf, mp_buf, n_win, d_win, acc_v,
                       semi, semg):
    c = lax.axis_index("c")
    s = lax.axis_index("s")
    w = s * 2 + c

    pltpu.sync_copy(starts_hbm, starts_v)
    pltpu.sync_copy(bases_hbm, bases_v)
    start = starts_v[pl.ds(w, 16)][0]
    end = starts_v[pl.ds(w + 1, 16)][0]
    lo_node = bases_v[pl.ds(w, 16)][0]
    hi_node = bases_v[pl.ds(w + 1, 16)][0]

    base0 = start & ~7
    nblk = (end - base0 + B - 1) // B

    zeros16 = jnp.zeros((16,), jnp.float32)

    def zero_win(r, _):
        for j in range(4):
            n_win[r, pl.ds(16 * j, 16)] = zeros16
            d_win[r, pl.ds(16 * j, 16)] = zeros16
        return 0

    lax.fori_loop(0, W, zero_win, 0)
    for j in range(8):
        acc_v[j, :] = zeros16

    def advance(cur_base, target):
        nfl = jnp.maximum((target - cur_base) // W, 0)

        def div_row(r, _):
            for j in range(4):
                nv = n_win[r, pl.ds(16 * j, 16)]
                dv = d_win[r, pl.ds(16 * j, 16)]
                n_win[r, pl.ds(16 * j, 16)] = nv / (dv + 1e-33)
            return 0

        def flush_i(i, cb):
            lax.fori_loop(0, W, div_row, 0)
            pltpu.sync_copy(n_win, out_hbm.at[pl.ds(cb, W)])
            lax.fori_loop(0, W, zero_win, 0)
            return cb + W

        return lax.fori_loop(0, nfl, flush_i, cur_base)

    NB = ((nblk + 1) // 2) * 2

    bufs = ((dst_v0, src_v0, order_v0, eeb0, xg0),
            (dst_v1, src_v1, order_v1, eeb1, xg1))

    def issue_idx(k, sslot):
        dstb, srcb, ordb, _, _ = bufs[sslot]
        base = pl.multiple_of(base0 + k * B, 8)
        pltpu.async_copy(dst_hbm.at[pl.ds(base, B)],
                         dstb.at[pl.ds(0, B)], semi)
        pltpu.async_copy(src_hbm.at[pl.ds(base, B)], srcb, semi)
        pltpu.async_copy(order_hbm.at[pl.ds(base, B)], ordb, semi)

    def wait_idx():
        for _ in range(2):
            pltpu.make_async_copy(src_hbm.at[pl.ds(0, B)],
                                  src_v0, semi).wait()
        pltpu.make_async_copy(dst_hbm.at[pl.ds(0, B)],
                              dst_v0.at[pl.ds(0, B)], semi).wait()

    def issue_gather(sslot):
        _, srcb, ordb, eebb, xgb = bufs[sslot]
        pltpu.async_copy(ee_hbm.at[ordb], eebb, semg)
        pltpu.async_copy(x_hbm.at[srcb], xgb, semg)

    def wait_gather():
        for _ in range(2):
            pltpu.make_async_copy(ee_hbm.at[pl.ds(0, B)],
                                  eeb0, semg).wait()

    def blk_fn(k, sslot, carry):
        dst_v, src_v, order_v, eeb, xg = bufs[sslot]
        base = base0 + k * B
        lo = jnp.maximum(start - base, 0)
        hi = jnp.minimum(end - base, B)
        wait_gather()

        def prep_fn(r, _):
            for j in range(4):
                xs = xg[r, pl.ds(16 * j, 16)]
                es = eeb[r, pl.ds(16 * j, 16)]
                msg = jnp.maximum(xs + es, 0.0) + EPS
                p = jnp.exp(msg - 60.0)
                p_buf[r, pl.ds(16 * j, 16)] = p
                mp_buf[r, pl.ds(16 * j, 16)] = msg * p
            return 0

        lax.fori_loop(0, B, prep_fn, 0)

        @pl.when(k + 1 < NB)
        def _():
            wait_idx()
            issue_gather(1 - sslot)

        def edge_fn(e, carry2):
            (cur_base, prev, d0, d1, d2, d3, n0, n1, n2, n3) = carry2
            d_e = dst_v[pl.ds(e, 16)][0]
            is_new = d_e != prev
            cur_base = advance(cur_base, d_e)
            keep = jnp.where(is_new, 0.0, 1.0)
            ds = [d0, d1, d2, d3]
            ns = [n0, n1, n2, n3]
            row = d_e - cur_base
            for j in range(4):
                p = p_buf[e, pl.ds(16 * j, 16)]
                mp = mp_buf[e, pl.ds(16 * j, 16)]
                d_j = ds[j] * keep + p
                n_j = ns[j] * keep + mp
                n_win[row, pl.ds(16 * j, 16)] = n_j
                d_win[row, pl.ds(16 * j, 16)] = d_j
                ds[j] = d_j
                ns[j] = n_j
            return (cur_base, d_e, ds[0], ds[1], ds[2], ds[3],
                    ns[0], ns[1], ns[2], ns[3])

        def chunk_fn(ch, carry2):
            e0 = ch * 16
            dvec = dst_v[pl.ds(e0, 16)]
            cur_base = carry2[0]
            in_range = jnp.logical_and(lo <= e0, e0 + 16 <= hi)
            fast = jnp.logical_and(in_range, dvec[15] < cur_base + W)

            def fast_fn(args):
                cur_base, prev = args
                ds = [acc_v[j, :] for j in range(4)]
                ns = [acc_v[4 + j, :] for j in range(4)]
                pr = prev
                for i in range(16):
                    d_e = dvec[i]
                    keep = jnp.where(d_e == pr, 1.0, 0.0)
                    row = d_e - cur_base
                    for j in range(4):
                        p = p_buf[e0 + i, pl.ds(16 * j, 16)]
                        mp = mp_buf[e0 + i, pl.ds(16 * j, 16)]
                        d_j = ds[j] * keep + p
                        n_j = ns[j] * keep + mp
                        n_win[row, pl.ds(16 * j, 16)] = n_j
                        d_win[row, pl.ds(16 * j, 16)] = d_j
                        ds[j] = d_j
                        ns[j] = n_j
                    pr = d_e
                for j in range(4):
                    acc_v[j, :] = ds[j]
                    acc_v[4 + j, :] = ns[j]
                return (cur_base, pr)

            def slow_fn(args):
                cur_base, prev = args
                lo_e = jnp.maximum(lo, e0)
                hi_e = jnp.minimum(hi, e0 + 16)
                full = (cur_base, prev) + tuple(
                    acc_v[j, :] for j in range(8))
                full = lax.fori_loop(lo_e, hi_e, edge_fn, full)
                for j in range(8):
                    acc_v[j, :] = full[2 + j]
                return (full[0], full[1])

            return lax.cond(fast, fast_fn, slow_fn, carry2)

        carry = lax.fori_loop(0, B // 16, chunk_fn, carry)

        @pl.when(k + 2 < NB)
        def _():
            issue_idx(k + 2, sslot)

        return carry

    def pair_fn(k2, carry):
        carry = blk_fn(2 * k2, 0, carry)
        carry = blk_fn(2 * k2 + 1, 1, carry)
        return carry

    init = (lo_node, jnp.int32(-1))

    @pl.when(NB > 0)
    def _():
        issue_idx(0, 0)
        wait_idx()
        issue_gather(0)
        issue_idx(1, 1)

    carry = lax.fori_loop(0, NB // 2, pair_fn, init)
    cur_base = carry[0]

    cur_base = advance(cur_base, hi_node)
    nrows = hi_node - cur_base

    def tail_div(r, _):
        for j in range(4):
            nv = n_win[r, pl.ds(16 * j, 16)]
            dv = d_win[r, pl.ds(16 * j, 16)]
            n_win[r, pl.ds(16 * j, 16)] = nv / (dv + 1e-33)
        return 0

    lax.fori_loop(0, nrows, tail_div, 0)

    def tail_fire(r, _):
        pltpu.async_copy(n_win.at[pl.ds(r, 1)],
                         out_hbm.at[pl.ds(cur_base + r, 1)], semg)
        return 0

    def tail_drain(r, _):
        pltpu.make_async_copy(n_win.at[pl.ds(0, 1)],
                              out_hbm.at[pl.ds(cur_base, 1)], semg).wait()
        return 0

    lax.fori_loop(0, nrows, tail_fire, 0)
    lax.fori_loop(0, nrows, tail_drain, 0)


def _sc_aggregate(x, ee, dst_p, src_p, order_p, starts, bases):
    mesh = plsc.VectorSubcoreMesh(core_axis_name="c", subcore_axis_name="s")
    f = pl.kernel(
        _sc_aggregate_body,
        out_type=jax.ShapeDtypeStruct((N, 64), jnp.float32),
        mesh=mesh,
        compiler_params=pltpu.CompilerParams(use_tc_tiling_on_sc=False),
        scratch_types=[
            pltpu.VMEM((48,), jnp.int32),
            pltpu.VMEM((48,), jnp.int32),
            pltpu.VMEM((B + 16,), jnp.int32),
            pltpu.VMEM((B,), jnp.int32),
            pltpu.VMEM((B,), jnp.int32),
            pltpu.VMEM((B + 16,), jnp.int32),
            pltpu.VMEM((B,), jnp.int32),
            pltpu.VMEM((B,), jnp.int32),
            pltpu.VMEM((B, 64), jnp.float32),
            pltpu.VMEM((B, 64), jnp.float32),
            pltpu.VMEM((B, 64), jnp.float32),
            pltpu.VMEM((B, 64), jnp.float32),
            pltpu.VMEM((B, 64), jnp.float32),
            pltpu.VMEM((B, 64), jnp.float32),
            pltpu.VMEM((W, 64), jnp.float32),
            pltpu.VMEM((W, 64), jnp.float32),
            pltpu.VMEM((8, 16), jnp.float32),
            pltpu.SemaphoreType.DMA,
            pltpu.SemaphoreType.DMA,
        ],
    )
    return f(x, ee, dst_p, src_p, order_p, starts, bases)


def kernel(node_features, edge_features, edge_links, params):
    src = edge_links[0]
    dst = edge_links[1]

    order = jnp.argsort(dst).astype(jnp.int32)
    dst_s = jnp.take(dst, order)
    src_s = jnp.take(src, order)

    pad = EPAD - E
    dst_p = jnp.concatenate([dst_s, jnp.full((pad,), N, jnp.int32)])
    src_p = jnp.concatenate([src_s, jnp.zeros((pad,), jnp.int32)])
    order_p = jnp.concatenate([order, jnp.zeros((pad,), jnp.int32)])

    nominal = (jnp.arange(1, NW) * E) // NW
    dvals = jnp.take(dst_s, nominal)
    starts_mid = jnp.searchsorted(dst_s, dvals, side="left").astype(jnp.int32)
    starts = jnp.concatenate(
        [jnp.zeros((1,), jnp.int32), starts_mid,
         jnp.full((1,), E, jnp.int32), jnp.zeros((48 - NW - 1,), jnp.int32)])
    base_mid = jnp.take(dst_s, starts_mid)
    bases = jnp.concatenate(
        [jnp.zeros((1,), jnp.int32), base_mid,
         jnp.full((1,), N, jnp.int32), jnp.zeros((48 - NW - 1,), jnp.int32)])

    x = _encoder3(node_features, params["node_enc"], rows=2000)
    ee = _encoder3(edge_features, params["edge_enc"], rows=3200)

    for bi, p in enumerate(params["mp"]):
        aggr = _sc_aggregate(x, ee, dst_p, src_p, order_p, starts, bases)
        if bi < 2:
            x = _block_mlp(aggr, x, p)
        else:
            return _block3_head(aggr, x, p, params["pi"])
